# Initial kernel scaffold; baseline (speedup 1.0000x reference)
import jax
import jax.numpy as jnp
from jax.experimental import pallas as pl


def kernel(features, codebook, lmbda):
    raise NotImplementedError("write your pallas kernel here")


# speedup vs baseline: 45.0331x; 45.0331x over previous
import functools

import jax
import jax.numpy as jnp
from jax.experimental import pallas as pl
from jax.experimental.pallas import tpu as pltpu


def _dpdp_body(lam_ref, f_ref, c_ref, units_ref, pc_ref, beta_s, gamma_s):
    f = f_ref[:]
    c = c_ref[:]
    T = f.shape[0]
    K = c.shape[0]
    lam = lam_ref[0]

    fn2 = jnp.sum(f * f, axis=1, keepdims=True)
    cn2 = jnp.sum(c * c, axis=1, keepdims=True).T
    g = jax.lax.dot_general(
        f, c, (((1,), (1,)), ((), ())),
        preferred_element_type=jnp.float32,
        precision=jax.lax.Precision.HIGHEST,
    )
    d2 = jnp.maximum(fn2 + cn2 - 2.0 * g, 0.0)

    mu = jnp.sum(d2) / jnp.float32(T * K)
    x = d2 - mu
    sh = 1
    while sh < T:
        x = x + jnp.concatenate([jnp.zeros((sh, K), jnp.float32), x[:-sh]], axis=0)
        sh *= 2
    pc_ref[:] = x

    kiota = jax.lax.broadcasted_iota(jnp.int32, (1, K), 1)

    row1 = pc_ref[pl.ds(0, 1), :]
    rmin1 = jnp.min(row1)
    k1 = jnp.min(jnp.where(row1 == rmin1, kiota, K))
    beta_s[0] = jnp.int32(0)
    gamma_s[0] = k1
    a0 = rmin1 + lam

    m0 = jnp.zeros((1, K), jnp.float32)
    marg0 = jnp.zeros((1, K), jnp.int32)

    def dp_step(t, carry):
        m, marg, a_prev = carry
        pj = pc_ref[pl.ds(t - 2, 1), :]
        pt = pc_ref[pl.ds(t - 1, 1), :]
        cand = a_prev - pj
        upd = cand < m
        marg = jnp.where(upd, t - 1, marg)
        m = jnp.where(upd, cand, m)
        row = pt + m
        rmin = jnp.min(row)
        kstar = jnp.min(jnp.where(row == rmin, kiota, K))
        beta_s[t - 1] = jnp.min(jnp.where(kiota == kstar, marg, jnp.int32(2**30)))
        gamma_s[t - 1] = kstar
        return (m, marg, rmin + lam)

    jax.lax.fori_loop(2, T + 1, dp_step, (m0, marg0, a0), unroll=False)

    uiota = jax.lax.broadcasted_iota(jnp.int32, (1, T), 1)

    def bt_cond(carry):
        _, idx = carry
        return idx > 0

    def bt_step(carry):
        units, idx = carry
        b = beta_s[idx - 1]
        gm = gamma_s[idx - 1]
        units = jnp.where((uiota >= b) & (uiota < idx), gm, units)
        return (units, b)

    units0 = jnp.zeros((1, T), jnp.int32)
    units, _ = jax.lax.while_loop(bt_cond, bt_step, (units0, jnp.int32(T)))
    units_ref[:] = units


def _dpdp_units(features, codebook, lam_arr):
    T, _ = features.shape
    K = codebook.shape[0]
    return pl.pallas_call(
        _dpdp_body,
        out_shape=jax.ShapeDtypeStruct((1, T), jnp.int32),
        in_specs=[
            pl.BlockSpec(memory_space=pltpu.SMEM),
            pl.BlockSpec(memory_space=pltpu.VMEM),
            pl.BlockSpec(memory_space=pltpu.VMEM),
        ],
        out_specs=pl.BlockSpec(memory_space=pltpu.VMEM),
        scratch_shapes=[
            pltpu.VMEM((T, K), jnp.float32),
            pltpu.SMEM((T,), jnp.int32),
            pltpu.SMEM((T,), jnp.int32),
        ],
    )(lam_arr, features, codebook)


def kernel(features, codebook, lmbda):
    lam_arr = jnp.reshape(jnp.asarray(lmbda, jnp.float32), (1,))
    units2d = _dpdp_units(features, codebook, lam_arr)
    units = units2d[0]
    indices = jnp.asarray(units, dtype=jnp.int64)
    quantized_features = jnp.take(codebook, units, axis=0)
    quantized_features_st = features - jax.lax.stop_gradient(
        features - quantized_features)
    return (quantized_features_st, indices)


# speedup vs baseline: 125.8722x; 2.7951x over previous
import functools

import jax
import jax.numpy as jnp
from jax.experimental import pallas as pl
from jax.experimental.pallas import tpu as pltpu


def _dpdp_body(lam_ref, f_ref, c_ref, units_ref, pc_ref, a_vm, beta_vm, gamma_vm):
    f = f_ref[:]
    c = c_ref[:]
    T = f.shape[0]
    K = c.shape[0]
    lam = lam_ref[0]

    fn2 = jnp.sum(f * f, axis=1, keepdims=True)
    cn2 = jnp.sum(c * c, axis=1, keepdims=True).T
    g = jax.lax.dot_general(
        f, c, (((1,), (1,)), ((), ())),
        preferred_element_type=jnp.float32,
        precision=jax.lax.Precision.HIGHEST,
    )
    d2 = jnp.maximum(fn2 + cn2 - 2.0 * g, 0.0)

    mu = jnp.sum(d2) / jnp.float32(T * K)
    x = d2 - mu
    sh = 1
    while sh < T:
        x = x + jnp.concatenate([jnp.zeros((sh, K), jnp.float32), x[:-sh]], axis=0)
        sh *= 2
    pc_ref[:] = x

    row1 = pc_ref[pl.ds(0, 1), :]
    a1 = jnp.min(row1) + lam
    a_vm[pl.ds(0, 1), :] = jnp.reshape(a1, (1, 1))

    def dp_step(t, carry):
        m, a_prev = carry
        pj = pc_ref[pl.ds(t - 2, 1), :]
        pt = pc_ref[pl.ds(t - 1, 1), :]
        m = jnp.minimum(m, a_prev - pj)
        a_t = jnp.min(pt + m) + lam
        a_vm[pl.ds(t - 1, 1), :] = jnp.reshape(a_t, (1, 1))
        return (m, a_t)

    m0 = jnp.zeros((1, K), jnp.float32)
    jax.lax.fori_loop(2, T + 1, dp_step, (m0, a1), unroll=False)

    pc = pc_ref[:]
    avec = a_vm[:]
    qfull = avec - pc
    q = jnp.concatenate([jnp.zeros((1, K), jnp.float32), qfull[:-1]], axis=0)
    mv = q
    ji = jax.lax.broadcasted_iota(jnp.int32, (T, K), 0)
    inf = jnp.float32(jnp.inf)
    sh = 1
    while sh < T:
        pv = jnp.concatenate([jnp.full((sh, K), inf, jnp.float32), mv[:-sh]], axis=0)
        pj = jnp.concatenate([jnp.zeros((sh, K), jnp.int32), ji[:-sh]], axis=0)
        take = pv <= mv
        mv = jnp.where(take, pv, mv)
        ji = jnp.where(take, pj, ji)
        sh *= 2
    r = pc + mv
    rmin = jnp.min(r, axis=1, keepdims=True)
    kiota = jax.lax.broadcasted_iota(jnp.int32, (1, K), 1)
    gam = jnp.min(jnp.where(r == rmin, kiota, K), axis=1, keepdims=True)
    bet = jnp.min(jnp.where(kiota == gam, ji, jnp.int32(2**30)),
                  axis=1, keepdims=True)
    beta_vm[:] = bet
    gamma_vm[:] = gam

    uiota = jax.lax.broadcasted_iota(jnp.int32, (1, T), 1)

    def bt_cond(carry):
        _, idx = carry
        return idx > 0

    def bt_step(carry):
        units, idx = carry
        b = jnp.min(beta_vm[pl.ds(idx - 1, 1), :])
        gm = jnp.min(gamma_vm[pl.ds(idx - 1, 1), :])
        units = jnp.where((uiota >= b) & (uiota < idx), gm, units)
        return (units, b)

    units0 = jnp.zeros((1, T), jnp.int32)
    units, _ = jax.lax.while_loop(bt_cond, bt_step, (units0, jnp.int32(T)))
    units_ref[:] = units


def _dpdp_units(features, codebook, lam_arr):
    T, _ = features.shape
    K = codebook.shape[0]
    return pl.pallas_call(
        _dpdp_body,
        out_shape=jax.ShapeDtypeStruct((1, T), jnp.int32),
        in_specs=[
            pl.BlockSpec(memory_space=pltpu.SMEM),
            pl.BlockSpec(memory_space=pltpu.VMEM),
            pl.BlockSpec(memory_space=pltpu.VMEM),
        ],
        out_specs=pl.BlockSpec(memory_space=pltpu.VMEM),
        scratch_shapes=[
            pltpu.VMEM((T, K), jnp.float32),
            pltpu.VMEM((T, 1), jnp.float32),
            pltpu.VMEM((T, 1), jnp.int32),
            pltpu.VMEM((T, 1), jnp.int32),
        ],
    )(lam_arr, features, codebook)


def kernel(features, codebook, lmbda):
    lam_arr = jnp.reshape(jnp.asarray(lmbda, jnp.float32), (1,))
    units2d = _dpdp_units(features, codebook, lam_arr)
    units = units2d[0]
    indices = jnp.asarray(units, dtype=jnp.int64)
    quantized_features = jnp.take(codebook, units, axis=0)
    quantized_features_st = features - jax.lax.stop_gradient(
        features - quantized_features)
    return (quantized_features_st, indices)
